# baseline (speedup 1.0000x reference)
import functools

import jax
import jax.numpy as jnp
from jax import lax
from jax.experimental import pallas as pl
from jax.experimental.pallas import tpu as pltpu
from jax.experimental.pallas import tpu_sc as plsc

VOCAB = 1000000
DIM = 64
BATCH = 16384
HIST = 50
B = BATCH * HIST

_info = plsc.get_sparse_core_info()
NC = _info.num_cores
NS = _info.num_subcores
NW = NC * NS

CH = 128
B_PER_W = B // NW
NROWS = B_PER_W // CH
K = 2
SB = K * CH
NSC = NROWS // K

_mesh = plsc.VectorSubcoreMesh(core_axis_name="c", subcore_axis_name="s")


@functools.partial(
    pl.kernel,
    mesh=_mesh,
    out_type=jax.ShapeDtypeStruct((B, 2, DIM), jnp.float32),
    scratch_types=[
        pltpu.VMEM((NROWS, CH), jnp.int32),
        pltpu.VMEM((SB, 2, DIM), jnp.float32),
        pltpu.VMEM((SB, 2, DIM), jnp.float32),
        pltpu.SemaphoreType.DMA,
        pltpu.SemaphoreType.DMA,
    ],
    compiler_params=pltpu.CompilerParams(use_tc_tiling_on_sc=False),
)
def _embed_lookup(idx_hbm, table_hbm, out_hbm, idx_v, buf0, buf1, sem0, sem1):
    wid = lax.axis_index("s") * NC + lax.axis_index("c")
    base = wid * B_PER_W

    pltpu.sync_copy(idx_hbm.at[wid], idx_v)

    def issue(buf, sem, sc):
        for k in range(K):
            pltpu.async_copy(
                table_hbm.at[idx_v.at[sc * K + k]],
                buf.at[pl.ds(k * CH, CH)],
                sem,
            )

    def drain(buf, sem):
        pltpu.make_async_copy(table_hbm.at[pl.ds(0, SB)], buf, sem).wait()

    issue(buf0, sem0, 0)
    issue(buf1, sem1, 1)

    def outer(o, carry):
        for p, (buf, sem) in enumerate(((buf0, sem0), (buf1, sem1))):
            sc = o * 2 + p
            drain(buf, sem)
            pltpu.sync_copy(buf, out_hbm.at[pl.ds(base + sc * SB, SB)])

            @pl.when(sc + 2 < NSC)
            def _():
                issue(buf, sem, sc + 2)

        return carry

    lax.fori_loop(0, NSC // 2, outer, 0)


def kernel(tokens, table):
    tblp = jnp.pad(table, ((0, 0), (0, DIM))).reshape(VOCAB, 2, DIM)
    idx3 = tokens.reshape(NW, NROWS, CH)
    out3 = _embed_lookup(idx3, tblp)
    return out3.reshape(B, 2 * DIM)[:, :DIM]


# speedup vs baseline: 4.5125x; 4.5125x over previous
import functools

import jax
import jax.numpy as jnp
from jax import lax
from jax.experimental import pallas as pl
from jax.experimental.pallas import tpu as pltpu
from jax.experimental.pallas import tpu_sc as plsc

VOCAB = 1000000
DIM = 64
BATCH = 16384
HIST = 50
B = BATCH * HIST

_info = plsc.get_sparse_core_info()
NC = _info.num_cores
NS = _info.num_subcores
NW = NC * NS

CH = 256
B_PER_W = B // NW
NROWS = B_PER_W // CH
K = 2
SB = K * CH
NSC = NROWS // K

_mesh = plsc.VectorSubcoreMesh(core_axis_name="c", subcore_axis_name="s")


@functools.partial(
    pl.kernel,
    mesh=_mesh,
    out_type=jax.ShapeDtypeStruct((B, DIM), jnp.float32),
    scratch_types=[
        pltpu.VMEM((NROWS, CH), jnp.int32),
        pltpu.VMEM((SB, DIM), jnp.float32),
        pltpu.VMEM((SB, DIM), jnp.float32),
        pltpu.SemaphoreType.DMA,
        pltpu.SemaphoreType.DMA,
    ],
    compiler_params=pltpu.CompilerParams(use_tc_tiling_on_sc=False),
)
def _embed_lookup(idx_hbm, table_hbm, out_hbm, idx_v, buf0, buf1, sem0, sem1):
    wid = lax.axis_index("s") * NC + lax.axis_index("c")
    base = wid * B_PER_W

    pltpu.sync_copy(idx_hbm.at[wid], idx_v)

    def issue(buf, sem, sc):
        for k in range(K):
            pltpu.async_copy(
                table_hbm.at[idx_v.at[sc * K + k]],
                buf.at[pl.ds(k * CH, CH)],
                sem,
            )

    def drain(buf, sem):
        pltpu.make_async_copy(table_hbm.at[pl.ds(0, SB)], buf, sem).wait()

    issue(buf0, sem0, 0)
    issue(buf1, sem1, 1)

    def outer(o, carry):
        for p, (buf, sem) in enumerate(((buf0, sem0), (buf1, sem1))):
            sc = o * 2 + p
            drain(buf, sem)
            pltpu.sync_copy(buf, out_hbm.at[pl.ds(base + sc * SB, SB)])

            @pl.when(sc + 2 < NSC)
            def _():
                issue(buf, sem, sc + 2)

        return carry

    lax.fori_loop(0, NSC // 2, outer, 0)


def kernel(tokens, table):
    idx3 = tokens.reshape(NW, NROWS, CH)
    return _embed_lookup(idx3, table)


# speedup vs baseline: 5.9992x; 1.3295x over previous
import functools

import jax
import jax.numpy as jnp
from jax import lax
from jax.experimental import pallas as pl
from jax.experimental.pallas import tpu as pltpu
from jax.experimental.pallas import tpu_sc as plsc

VOCAB = 1000000
DIM = 64
BATCH = 16384
HIST = 50
B = BATCH * HIST

_info = plsc.get_sparse_core_info()
NC = _info.num_cores
NS = _info.num_subcores
NW = NC * NS

CH = 256
B_PER_W = B // NW
NROWS = B_PER_W // CH
K = 2
SB = K * CH
NSC = NROWS // K

_mesh = plsc.VectorSubcoreMesh(core_axis_name="c", subcore_axis_name="s")


@functools.partial(
    pl.kernel,
    mesh=_mesh,
    out_type=jax.ShapeDtypeStruct((B, 2 * DIM), jnp.float32),
    scratch_types=[
        pltpu.VMEM((NROWS, CH), jnp.int32),
        pltpu.VMEM((SB, DIM), jnp.float32),
        pltpu.VMEM((SB, DIM), jnp.float32),
        pltpu.SemaphoreType.DMA,
        pltpu.SemaphoreType.DMA,
    ],
    compiler_params=pltpu.CompilerParams(use_tc_tiling_on_sc=False),
)
def _embed_lookup(idx_hbm, table_hbm, out_hbm, idx_v, buf0, buf1, sem0, sem1):
    wid = lax.axis_index("s") * NC + lax.axis_index("c")
    base = wid * B_PER_W

    pltpu.sync_copy(idx_hbm.at[wid], idx_v)

    def issue(buf, sem, sc):
        for k in range(K):
            pltpu.async_copy(
                table_hbm.at[idx_v.at[sc * K + k]],
                buf.at[pl.ds(k * CH, CH)],
                sem,
            )

    def drain(buf, sem):
        pltpu.make_async_copy(table_hbm.at[pl.ds(0, SB)], buf, sem).wait()

    issue(buf0, sem0, 0)
    issue(buf1, sem1, 1)

    def outer(o, carry):
        for p, (buf, sem) in enumerate(((buf0, sem0), (buf1, sem1))):
            sc = o * 2 + p
            drain(buf, sem)
            pltpu.sync_copy(
                buf, out_hbm.at[pl.ds(base + sc * SB, SB), pl.ds(0, DIM)])

            @pl.when(sc + 2 < NSC)
            def _():
                issue(buf, sem, sc + 2)

        return carry

    lax.fori_loop(0, NSC // 2, outer, 0)


def kernel(tokens, table):
    idx3 = tokens.reshape(NW, NROWS, CH)
    return _embed_lookup(idx3, table)[:, :DIM]
